# baseline (speedup 1.0000x reference)
import jax
import jax.numpy as jnp
from jax.experimental import pallas as pl
from jax.experimental.pallas import tpu as pltpu

B = 16
D = 64
S = 32 * 32
K = 1024
COMMITMENT_COST = 0.25


BB = 2
M = BB * S


def _vq_kernel(z_ref, w_ref, out_ref, loss_ref, perp_ref, counts_ref, acc_ref):
    b = pl.program_id(0)

    w = w_ref[...]

    zs = [z_ref[i].reshape(D, S) for i in range(BB)]
    zt = jnp.concatenate([z_i.T for z_i in zs], axis=0)

    mm2 = jax.lax.dot_general(
        zt, w + w, (((1,), (1,)), ((), ())),
        preferred_element_type=jnp.float32)
    zsq = jnp.sum(zt * zt, axis=1, keepdims=True)
    wsq = jnp.sum(w * w, axis=1).reshape(1, K)
    dist = (zsq + wsq) - mm2

    iota = jax.lax.broadcasted_iota(jnp.int32, (M, K), 1)
    mval = jnp.min(dist, axis=1, keepdims=True)
    idx = jnp.min(jnp.where(dist == mval, iota, K), axis=1).reshape(M, 1)

    onehot = (iota == idx).astype(jnp.float32)

    q = jax.lax.dot_general(
        w, onehot, (((0,), (1,)), ((), ())),
        preferred_element_type=jnp.float32)

    sq = jnp.float32(0.0)
    for i in range(BB):
        z_i = zs[i]
        diff = q[:, i * S:(i + 1) * S] - z_i
        out_ref[i] = (z_i + diff).reshape(D, 32, 32)
        sq += jnp.sum(diff * diff)

    counts_b = jax.lax.dot_general(
        jnp.ones((8, M), jnp.float32), onehot, (((1,), (0,)), ((), ())),
        preferred_element_type=jnp.float32)

    @pl.when(b == 0)
    def _init():
        acc_ref[0, 0] = sq
        counts_ref[...] = counts_b

    @pl.when(b > 0)
    def _acc():
        acc_ref[0, 0] += sq
        counts_ref[...] += counts_b

    @pl.when(b == B // BB - 1)
    def _fin():
        n = jnp.float32(B * S)
        loss = (1.0 + COMMITMENT_COST) * acc_ref[0, 0] / (n * D)
        loss_ref[...] = loss.reshape(1, 1)
        probs = counts_ref[0:1, :] / n
        ent = -jnp.sum(probs * jnp.log(probs + 1e-10))
        perp_ref[...] = jnp.exp(ent).reshape(1, 1)


def kernel(z, W):
    q, loss, perp = pl.pallas_call(
        _vq_kernel,
        grid=(B // BB,),
        in_specs=[
            pl.BlockSpec((BB, D, 32, 32), lambda b: (b, 0, 0, 0)),
            pl.BlockSpec((K, D), lambda b: (0, 0)),
        ],
        out_specs=[
            pl.BlockSpec((BB, D, 32, 32), lambda b: (b, 0, 0, 0)),
            pl.BlockSpec((1, 1), lambda b: (0, 0)),
            pl.BlockSpec((1, 1), lambda b: (0, 0)),
        ],
        out_shape=[
            jax.ShapeDtypeStruct((B, D, 32, 32), jnp.float32),
            jax.ShapeDtypeStruct((1, 1), jnp.float32),
            jax.ShapeDtypeStruct((1, 1), jnp.float32),
        ],
        scratch_shapes=[
            pltpu.VMEM((8, K), jnp.float32),
            pltpu.SMEM((1, 1), jnp.float32),
        ],
    )(z, W)
    return (q, loss[0, 0], perp[0, 0])


# speedup vs baseline: 1.4755x; 1.4755x over previous
import jax
import jax.numpy as jnp
from jax.experimental import pallas as pl
from jax.experimental.pallas import tpu as pltpu

B = 16
D = 64
S = 32 * 32
K = 1024
COMMITMENT_COST = 0.25


BB = 4
M = BB * S


def _vq_kernel(z_ref, w_ref, out_ref, loss_ref, perp_ref, counts_ref, acc_ref):
    b = pl.program_id(0)

    w = w_ref[...]

    zt = jnp.concatenate([z_ref[i].T for i in range(BB)], axis=0)

    mm2 = jax.lax.dot_general(
        zt, w + w, (((1,), (1,)), ((), ())),
        preferred_element_type=jnp.float32)
    zsq = jnp.sum(zt * zt, axis=1, keepdims=True)
    wsq = jnp.sum(w * w, axis=1).reshape(1, K)
    dist = (zsq + wsq) - mm2

    iota = jax.lax.broadcasted_iota(jnp.int32, (M, K), 1)
    mval = jnp.min(dist, axis=1, keepdims=True)
    idx = jnp.min(jnp.where(dist == mval, iota, K), axis=1).reshape(M, 1)

    onehot = (iota == idx).astype(jnp.float32)

    q = jax.lax.dot_general(
        w, onehot, (((0,), (1,)), ((), ())),
        preferred_element_type=jnp.float32)

    sq = jnp.float32(0.0)
    for i in range(BB):
        z_i = z_ref[i]
        diff = q[:, i * S:(i + 1) * S] - z_i
        out_ref[i] = z_i + diff
        sq += jnp.sum(diff * diff)

    counts_b = jax.lax.dot_general(
        jnp.ones((8, M), jnp.float32), onehot, (((1,), (0,)), ((), ())),
        preferred_element_type=jnp.float32)

    @pl.when(b == 0)
    def _init():
        acc_ref[0, 0] = sq
        counts_ref[...] = counts_b

    @pl.when(b > 0)
    def _acc():
        acc_ref[0, 0] += sq
        counts_ref[...] += counts_b

    @pl.when(b == B // BB - 1)
    def _fin():
        n = jnp.float32(B * S)
        loss = (1.0 + COMMITMENT_COST) * acc_ref[0, 0] / (n * D)
        loss_ref[...] = loss.reshape(1, 1)
        probs = counts_ref[0:1, :] / n
        ent = -jnp.sum(probs * jnp.log(probs + 1e-10))
        perp_ref[...] = jnp.exp(ent).reshape(1, 1)


def kernel(z, W):
    z3 = z.reshape(B, D, S)
    q, loss, perp = pl.pallas_call(
        _vq_kernel,
        grid=(B // BB,),
        in_specs=[
            pl.BlockSpec((BB, D, S), lambda b: (b, 0, 0)),
            pl.BlockSpec((K, D), lambda b: (0, 0)),
        ],
        out_specs=[
            pl.BlockSpec((BB, D, S), lambda b: (b, 0, 0)),
            pl.BlockSpec((1, 1), lambda b: (0, 0)),
            pl.BlockSpec((1, 1), lambda b: (0, 0)),
        ],
        out_shape=[
            jax.ShapeDtypeStruct((B, D, S), jnp.float32),
            jax.ShapeDtypeStruct((1, 1), jnp.float32),
            jax.ShapeDtypeStruct((1, 1), jnp.float32),
        ],
        scratch_shapes=[
            pltpu.VMEM((8, K), jnp.float32),
            pltpu.SMEM((1, 1), jnp.float32),
        ],
    )(z3, W)
    return (q.reshape(B, D, 32, 32), loss[0, 0], perp[0, 0])
